# baseline (speedup 1.0000x reference)
import functools

import jax
import jax.numpy as jnp
from jax import lax
from jax.experimental import pallas as pl
from jax.experimental.pallas import tpu as pltpu
from jax.experimental.pallas import tpu_sc as plsc

N = 10000
E = 320000
D_IN, H1, H2 = 128, 64, 32
EDGE_TYPES = ((0, 0), (0, 1), (1, 0), (1, 1))

NC, NS = 2, 16
NW = NC * NS
EPT = E // NW
EPTP = 10240
DUMMY = 10232
NPAD = 10240
SLICE = NPAD // NS
BLK = 1024
GRID = NPAD // BLK


NCH = EPTP // 128


@functools.cache
def _get_sc_hist():
    mesh = plsc.VectorSubcoreMesh(
        core_axis_name="c", subcore_axis_name="s",
        num_cores=NC, num_subcores=NS)
    return functools.partial(
        pl.kernel,
        out_type=jax.ShapeDtypeStruct((NC, 8, NPAD), jnp.float32),
        mesh=mesh,
        scratch_types=(
            [pltpu.VMEM((NCH, 128), jnp.int32),
             pltpu.VMEM((128,), jnp.float32)]
            + [pltpu.VMEM_SHARED((NPAD,), jnp.float32) for _ in range(8)]
        ),
        compiler_params=pltpu.CompilerParams(use_tc_tiling_on_sc=False),
        name="sc_hist",
    )(_sc_hist_body)


def _sc_hist_body(edges, zeros1, out, idxbuf, ones, *accs):
    cid = lax.axis_index("c")
    sid = lax.axis_index("s")
    slab = cid * NS + sid
    for k in range(8):
        ones[pl.ds(k * 16, 16)] = jnp.ones((16,), jnp.float32)
    for h in range(8):
        pltpu.sync_copy(zeros1.at[pl.ds(sid * SLICE, SLICE)],
                        accs[h].at[pl.ds(sid * SLICE, SLICE)])
    plsc.subcore_barrier()
    for h in range(8):
        et, r = divmod(h, 2)
        pltpu.sync_copy(edges.at[et, r, slab], idxbuf)

        def body(j, _, h=h):
            pltpu.sync_copy(ones, accs[h].at[idxbuf.at[j]], add=True)
            return 0

        lax.fori_loop(0, NCH, body, 0)
    plsc.subcore_barrier()
    for h in range(8):
        pltpu.sync_copy(accs[h].at[pl.ds(sid * SLICE, SLICE)],
                        out.at[cid, h, pl.ds(sid * SLICE, SLICE)])


@functools.cache
def _make_sc_spmm(w):
    nops = EPTP // 128
    mesh = plsc.VectorSubcoreMesh(
        core_axis_name="c", subcore_axis_name="s",
        num_cores=NC, num_subcores=NS)

    @functools.partial(
        pl.kernel,
        out_type=jax.ShapeDtypeStruct((4, NC, NPAD, w), jnp.float32),
        mesh=mesh,
        scratch_types=[
            pltpu.VMEM((nops, 128), jnp.int32),
            pltpu.VMEM((nops, 128), jnp.int32),
            pltpu.VMEM((2, 128, w), jnp.float32),
            pltpu.VMEM_SHARED((NPAD, w), jnp.float32),
            pltpu.SemaphoreType.DMA,
            pltpu.SemaphoreType.DMA,
        ],
        compiler_params=pltpu.CompilerParams(use_tc_tiling_on_sc=False),
        name=f"sc_spmm_{w}",
    )
    def spmm(htab, edges, zw, out, sidx, didx, rows, acc, gsem, ssem):
        cid = lax.axis_index("c")
        sid = lax.axis_index("s")
        slab = cid * NS + sid
        for et in range(4):
            pltpu.sync_copy(zw.at[pl.ds(sid * SLICE, SLICE)],
                            acc.at[pl.ds(sid * SLICE, SLICE)])
            pltpu.sync_copy(edges.at[et, 1, slab], sidx)
            pltpu.sync_copy(edges.at[et, 0, slab], didx)
            plsc.subcore_barrier()

            pltpu.async_copy(htab.at[et].at[sidx.at[0]], rows.at[0], gsem)

            def body(j, _, et=et):
                b = lax.rem(j, 2)
                pltpu.make_async_copy(
                    htab.at[et].at[sidx.at[j]], rows.at[b], gsem).wait()

                @pl.when(j >= 1)
                def _():
                    pltpu.make_async_copy(
                        rows.at[1 - b], acc.at[didx.at[j]], ssem).wait()

                @pl.when(j + 1 < nops)
                def _():
                    pltpu.async_copy(
                        htab.at[et].at[sidx.at[j + 1]], rows.at[1 - b], gsem)

                pltpu.async_copy(rows.at[b], acc.at[didx.at[j]], ssem,
                                 add=True)
                return 0

            lax.fori_loop(0, nops, body, 0)
            pltpu.make_async_copy(rows.at[0], acc.at[didx.at[0]], ssem).wait()
            plsc.subcore_barrier()
            pltpu.sync_copy(acc.at[pl.ds(sid * SLICE, SLICE)],
                            out.at[et, cid, pl.ds(sid * SLICE, SLICE)])
            plsc.subcore_barrier()

    return spmm


def _l2n(x):
    return x * lax.rsqrt(jnp.sum(x * x, axis=1, keepdims=True) + 1e-12)


def _p1_body(f0, f1, w00, w01, w10, w11, degp, htab, coef):
    c = lax.rsqrt(jnp.maximum(degp[0] + degp[1], 1.0))
    coef[...] = c
    feats = (f0[...], f1[...])
    ws = (w00[...], w01[...], w10[...], w11[...])
    for et, (_, j) in enumerate(EDGE_TYPES):
        h = jnp.dot(feats[j], ws[et], preferred_element_type=jnp.float32)
        htab[et] = h * c[2 * et + 1][:, None]


def _p2_body(acc1, coef, w00, w01, w10, w11, htab2):
    c = coef[...]
    m = []
    for et in range(4):
        s = acc1[et, 0] + acc1[et, 1]
        x = jax.nn.relu(s * c[2 * et][:, None])
        m.append(_l2n(x))
    hidden = (jax.nn.relu(m[0] + m[1]), jax.nn.relu(m[2] + m[3]))
    ws = (w00[...], w01[...], w10[...], w11[...])
    for et, (_, j) in enumerate(EDGE_TYPES):
        h = jnp.dot(hidden[j], ws[et], preferred_element_type=jnp.float32)
        htab2[et] = h * c[2 * et + 1][:, None]


def _p3_body(acc2, coef, o0, o1):
    c = coef[...]
    e = [jnp.zeros((BLK, H2), jnp.float32), jnp.zeros((BLK, H2), jnp.float32)]
    for et, (i, _) in enumerate(EDGE_TYPES):
        s = acc2[et, 0] + acc2[et, 1]
        e[i] = e[i] + _l2n(s * c[2 * et][:, None])
    o0[...] = e[0]
    o1[...] = e[1]


def _tc_p1(f0, f1, w00, w01, w10, w11, degp):
    wspec = pl.BlockSpec((D_IN, H1), lambda i: (0, 0))
    return pl.pallas_call(
        _p1_body,
        grid=(GRID,),
        in_specs=[
            pl.BlockSpec((BLK, D_IN), lambda i: (i, 0)),
            pl.BlockSpec((BLK, D_IN), lambda i: (i, 0)),
            wspec, wspec, wspec, wspec,
            pl.BlockSpec((NC, 8, BLK), lambda i: (0, 0, i)),
        ],
        out_specs=[
            pl.BlockSpec((4, BLK, H1), lambda i: (0, i, 0)),
            pl.BlockSpec((8, BLK), lambda i: (0, i)),
        ],
        out_shape=[
            jax.ShapeDtypeStruct((4, NPAD, H1), jnp.float32),
            jax.ShapeDtypeStruct((8, NPAD), jnp.float32),
        ],
    )(f0, f1, w00, w01, w10, w11, degp)


def _tc_p2(acc1, coef, w00, w01, w10, w11):
    wspec = pl.BlockSpec((H1, H2), lambda i: (0, 0))
    return pl.pallas_call(
        _p2_body,
        grid=(GRID,),
        in_specs=[
            pl.BlockSpec((4, NC, BLK, H1), lambda i: (0, 0, i, 0)),
            pl.BlockSpec((8, BLK), lambda i: (0, i)),
            wspec, wspec, wspec, wspec,
        ],
        out_specs=pl.BlockSpec((4, BLK, H2), lambda i: (0, i, 0)),
        out_shape=jax.ShapeDtypeStruct((4, NPAD, H2), jnp.float32),
    )(acc1, coef, w00, w01, w10, w11)


def _tc_p3(acc2, coef):
    return pl.pallas_call(
        _p3_body,
        grid=(GRID,),
        in_specs=[
            pl.BlockSpec((4, NC, BLK, H2), lambda i: (0, 0, i, 0)),
            pl.BlockSpec((8, BLK), lambda i: (0, i)),
        ],
        out_specs=[
            pl.BlockSpec((BLK, H2), lambda i: (i, 0)),
            pl.BlockSpec((BLK, H2), lambda i: (i, 0)),
        ],
        out_shape=[
            jax.ShapeDtypeStruct((NPAD, H2), jnp.float32),
            jax.ShapeDtypeStruct((NPAD, H2), jnp.float32),
        ],
    )(acc2, coef)


def kernel(feat_0, feat_1, edge_index_00, edge_index_01, edge_index_10,
           edge_index_11, W1_00, W2_00, W1_01, W2_01, W1_10, W2_10,
           W1_11, W2_11):
    edges = jnp.stack(
        [edge_index_00, edge_index_01, edge_index_10, edge_index_11])
    edges = edges.reshape(4, 2, NW, EPT)
    dummy = jnp.broadcast_to(
        N + jnp.arange(EPTP - EPT, dtype=jnp.int32), (4, 2, NW, EPTP - EPT))
    edges = jnp.concatenate([edges, dummy], axis=3)
    edges_a = edges.reshape(4, 2, NW, NCH, 128)

    zeros1 = jnp.zeros((NPAD,), jnp.float32)
    zw1 = jnp.zeros((NPAD, H1), jnp.float32)
    zw2 = jnp.zeros((NPAD, H2), jnp.float32)
    pad = ((0, NPAD - N), (0, 0))
    f0 = jnp.pad(feat_0, pad)
    f1 = jnp.pad(feat_1, pad)

    degp = _get_sc_hist()(edges_a, zeros1)
    htab1, coef = _tc_p1(f0, f1, W1_00, W1_01, W1_10, W1_11, degp)
    acc1 = _make_sc_spmm(H1)(htab1, edges_a, zw1)
    htab2 = _tc_p2(acc1, coef, W2_00, W2_01, W2_10, W2_11)
    acc2 = _make_sc_spmm(H2)(htab2, edges_a, zw2)
    e0, e1 = _tc_p3(acc2, coef)
    return jnp.concatenate([e0[:N], e1[:N]], axis=0)


# speedup vs baseline: 1.3968x; 1.3968x over previous
import functools

import jax
import jax.numpy as jnp
from jax import lax
from jax.experimental import pallas as pl
from jax.experimental.pallas import tpu as pltpu
from jax.experimental.pallas import tpu_sc as plsc

N = 10000
E = 320000
D_IN, H1, H2 = 128, 64, 32
EDGE_TYPES = ((0, 0), (0, 1), (1, 0), (1, 1))

NC, NS = 2, 16
NW = NC * NS
EPT = E // NW
EPTP = 10240
DUMMY = 10232
NPAD = 10240
SLICE = NPAD // NS
BLK = 1024
GRID = NPAD // BLK


NCH = EPTP // 128


@functools.cache
def _get_sc_hist():
    mesh = plsc.VectorSubcoreMesh(
        core_axis_name="c", subcore_axis_name="s",
        num_cores=NC, num_subcores=NS)
    return functools.partial(
        pl.kernel,
        out_type=jax.ShapeDtypeStruct((NC, 8, NPAD), jnp.float32),
        mesh=mesh,
        scratch_types=(
            [pltpu.VMEM((NCH, 128), jnp.int32),
             pltpu.VMEM((128,), jnp.float32),
             pltpu.SemaphoreType.DMA]
            + [pltpu.VMEM_SHARED((NPAD,), jnp.float32) for _ in range(8)]
        ),
        compiler_params=pltpu.CompilerParams(use_tc_tiling_on_sc=False),
        name="sc_hist",
    )(_sc_hist_body)


def _sc_hist_body(edges, zeros1, out, idxbuf, ones, ssem, *accs):
    cid = lax.axis_index("c")
    sid = lax.axis_index("s")
    slab = cid * NS + sid
    for k in range(8):
        ones[pl.ds(k * 16, 16)] = jnp.ones((16,), jnp.float32)
    for h in range(8):
        pltpu.sync_copy(zeros1.at[pl.ds(sid * SLICE, SLICE)],
                        accs[h].at[pl.ds(sid * SLICE, SLICE)])
    plsc.subcore_barrier()
    for h in range(8):
        et, r = divmod(h, 2)
        pltpu.sync_copy(edges.at[et, r, slab], idxbuf)

        def body(j, _, h=h):
            @pl.when(j >= 4)
            def _():
                pltpu.make_async_copy(
                    ones, accs[h].at[idxbuf.at[j]], ssem).wait()

            pltpu.async_copy(ones, accs[h].at[idxbuf.at[j]], ssem, add=True)
            return 0

        lax.fori_loop(0, NCH, body, 0)
        for _ in range(4):
            pltpu.make_async_copy(ones, accs[h].at[idxbuf.at[0]], ssem).wait()
    plsc.subcore_barrier()
    for h in range(8):
        pltpu.sync_copy(accs[h].at[pl.ds(sid * SLICE, SLICE)],
                        out.at[cid, h, pl.ds(sid * SLICE, SLICE)])


@functools.cache
def _make_sc_spmm(w):
    nops = EPTP // 128
    mesh = plsc.VectorSubcoreMesh(
        core_axis_name="c", subcore_axis_name="s",
        num_cores=NC, num_subcores=NS)

    @functools.partial(
        pl.kernel,
        out_type=jax.ShapeDtypeStruct((4, NC, NPAD, w), jnp.float32),
        mesh=mesh,
        scratch_types=[
            pltpu.VMEM((nops, 128), jnp.int32),
            pltpu.VMEM((nops, 128), jnp.int32),
            pltpu.VMEM((4, 128, w), jnp.float32),
            pltpu.VMEM_SHARED((NPAD, w), jnp.float32),
            pltpu.SemaphoreType.DMA,
            pltpu.SemaphoreType.DMA,
        ],
        compiler_params=pltpu.CompilerParams(use_tc_tiling_on_sc=False),
        name=f"sc_spmm_{w}",
    )
    def spmm(htab, edges, zw, out, sidx, didx, rows, acc, gsem, ssem):
        cid = lax.axis_index("c")
        sid = lax.axis_index("s")
        slab = cid * NS + sid
        for et in range(4):
            pltpu.sync_copy(zw.at[pl.ds(sid * SLICE, SLICE)],
                            acc.at[pl.ds(sid * SLICE, SLICE)])
            pltpu.sync_copy(edges.at[et, 1, slab], sidx)
            pltpu.sync_copy(edges.at[et, 0, slab], didx)
            plsc.subcore_barrier()

            pltpu.async_copy(htab.at[et].at[sidx.at[0]], rows.at[0], gsem)
            pltpu.async_copy(htab.at[et].at[sidx.at[1]], rows.at[1], gsem)

            def body(j, _, et=et):
                b = lax.rem(j, 4)
                pltpu.make_async_copy(
                    htab.at[et].at[sidx.at[j]], rows.at[b], gsem).wait()

                @pl.when(j >= 2)
                def _():
                    pltpu.make_async_copy(
                        rows.at[b], acc.at[didx.at[j]], ssem).wait()

                @pl.when(j + 2 < nops)
                def _():
                    pltpu.async_copy(htab.at[et].at[sidx.at[j + 2]],
                                     rows.at[lax.rem(j + 2, 4)], gsem)

                pltpu.async_copy(rows.at[b], acc.at[didx.at[j]], ssem,
                                 add=True)
                return 0

            lax.fori_loop(0, nops, body, 0)
            for _ in range(2):
                pltpu.make_async_copy(
                    rows.at[0], acc.at[didx.at[0]], ssem).wait()
            plsc.subcore_barrier()
            pltpu.sync_copy(acc.at[pl.ds(sid * SLICE, SLICE)],
                            out.at[et, cid, pl.ds(sid * SLICE, SLICE)])
            plsc.subcore_barrier()

    return spmm


def _l2n(x):
    return x * lax.rsqrt(jnp.sum(x * x, axis=1, keepdims=True) + 1e-12)


def _p1_body(f0, f1, w00, w01, w10, w11, degp, htab, coef):
    c = lax.rsqrt(jnp.maximum(degp[0] + degp[1], 1.0))
    coef[...] = c
    feats = (f0[...], f1[...])
    ws = (w00[...], w01[...], w10[...], w11[...])
    for et, (_, j) in enumerate(EDGE_TYPES):
        h = jnp.dot(feats[j], ws[et], preferred_element_type=jnp.float32)
        htab[et] = h * c[2 * et + 1][:, None]


def _p2_body(acc1, coef, w00, w01, w10, w11, htab2):
    c = coef[...]
    m = []
    for et in range(4):
        s = acc1[et, 0] + acc1[et, 1]
        x = jax.nn.relu(s * c[2 * et][:, None])
        m.append(_l2n(x))
    hidden = (jax.nn.relu(m[0] + m[1]), jax.nn.relu(m[2] + m[3]))
    ws = (w00[...], w01[...], w10[...], w11[...])
    for et, (_, j) in enumerate(EDGE_TYPES):
        h = jnp.dot(hidden[j], ws[et], preferred_element_type=jnp.float32)
        htab2[et] = h * c[2 * et + 1][:, None]


def _p3_body(acc2, coef, o0, o1):
    c = coef[...]
    e = [jnp.zeros((BLK, H2), jnp.float32), jnp.zeros((BLK, H2), jnp.float32)]
    for et, (i, _) in enumerate(EDGE_TYPES):
        s = acc2[et, 0] + acc2[et, 1]
        e[i] = e[i] + _l2n(s * c[2 * et][:, None])
    o0[...] = e[0]
    o1[...] = e[1]


def _tc_p1(f0, f1, w00, w01, w10, w11, degp):
    wspec = pl.BlockSpec((D_IN, H1), lambda i: (0, 0))
    return pl.pallas_call(
        _p1_body,
        grid=(GRID,),
        in_specs=[
            pl.BlockSpec((BLK, D_IN), lambda i: (i, 0)),
            pl.BlockSpec((BLK, D_IN), lambda i: (i, 0)),
            wspec, wspec, wspec, wspec,
            pl.BlockSpec((NC, 8, BLK), lambda i: (0, 0, i)),
        ],
        out_specs=[
            pl.BlockSpec((4, BLK, H1), lambda i: (0, i, 0)),
            pl.BlockSpec((8, BLK), lambda i: (0, i)),
        ],
        out_shape=[
            jax.ShapeDtypeStruct((4, NPAD, H1), jnp.float32),
            jax.ShapeDtypeStruct((8, NPAD), jnp.float32),
        ],
    )(f0, f1, w00, w01, w10, w11, degp)


def _tc_p2(acc1, coef, w00, w01, w10, w11):
    wspec = pl.BlockSpec((H1, H2), lambda i: (0, 0))
    return pl.pallas_call(
        _p2_body,
        grid=(GRID,),
        in_specs=[
            pl.BlockSpec((4, NC, BLK, H1), lambda i: (0, 0, i, 0)),
            pl.BlockSpec((8, BLK), lambda i: (0, i)),
            wspec, wspec, wspec, wspec,
        ],
        out_specs=pl.BlockSpec((4, BLK, H2), lambda i: (0, i, 0)),
        out_shape=jax.ShapeDtypeStruct((4, NPAD, H2), jnp.float32),
    )(acc1, coef, w00, w01, w10, w11)


def _tc_p3(acc2, coef):
    return pl.pallas_call(
        _p3_body,
        grid=(GRID,),
        in_specs=[
            pl.BlockSpec((4, NC, BLK, H2), lambda i: (0, 0, i, 0)),
            pl.BlockSpec((8, BLK), lambda i: (0, i)),
        ],
        out_specs=[
            pl.BlockSpec((BLK, H2), lambda i: (i, 0)),
            pl.BlockSpec((BLK, H2), lambda i: (i, 0)),
        ],
        out_shape=[
            jax.ShapeDtypeStruct((NPAD, H2), jnp.float32),
            jax.ShapeDtypeStruct((NPAD, H2), jnp.float32),
        ],
    )(acc2, coef)


def kernel(feat_0, feat_1, edge_index_00, edge_index_01, edge_index_10,
           edge_index_11, W1_00, W2_00, W1_01, W2_01, W1_10, W2_10,
           W1_11, W2_11):
    edges = jnp.stack(
        [edge_index_00, edge_index_01, edge_index_10, edge_index_11])
    edges = edges.reshape(4, 2, NW, EPT)
    dummy = jnp.broadcast_to(
        N + jnp.arange(EPTP - EPT, dtype=jnp.int32), (4, 2, NW, EPTP - EPT))
    edges = jnp.concatenate([edges, dummy], axis=3)
    edges_a = edges.reshape(4, 2, NW, NCH, 128)

    zeros1 = jnp.zeros((NPAD,), jnp.float32)
    zw1 = jnp.zeros((NPAD, H1), jnp.float32)
    zw2 = jnp.zeros((NPAD, H2), jnp.float32)
    pad = ((0, NPAD - N), (0, 0))
    f0 = jnp.pad(feat_0, pad)
    f1 = jnp.pad(feat_1, pad)

    degp = _get_sc_hist()(edges_a, zeros1)
    htab1, coef = _tc_p1(f0, f1, W1_00, W1_01, W1_10, W1_11, degp)
    acc1 = _make_sc_spmm(H1)(htab1, edges_a, zw1)
    htab2 = _tc_p2(acc1, coef, W2_00, W2_01, W2_10, W2_11)
    acc2 = _make_sc_spmm(H2)(htab2, edges_a, zw2)
    e0, e1 = _tc_p3(acc2, coef)
    return jnp.concatenate([e0[:N], e1[:N]], axis=0)
